# baseline (speedup 1.0000x reference)
import jax
import jax.numpy as jnp
from jax import lax
from jax.experimental import pallas as pl
from jax.experimental.pallas import tpu as pltpu

D = 1024
H = 16
HD = D // H
B, S = 2, 1024
N = B * S
PAIR = 2
GH = H // PAIR
BQ = 256
SCALE = 1.0 / 32.0

_CONTRACT_LAST = (((1,), (1,)), ((), ()))
_F32 = jnp.float32
_BF16 = jnp.bfloat16


def _mha_kernel(x_ref, w_ref, b_ref, wq_ref, wk_ref, o_ref):
    x = x_ref[...]
    w = w_ref[...]
    flat2 = lax.dot_general(x, w, _CONTRACT_LAST,
                            preferred_element_type=_F32) + b_ref[0]
    flat2 = flat2.astype(_BF16)
    for p in range(PAIR):
        flat = flat2[:, p * HD:(p + 1) * HD]
        q = jnp.dot(flat, wq_ref[p], preferred_element_type=_F32).astype(_BF16)
        k = jnp.dot(flat, wk_ref[p], preferred_element_type=_F32).astype(_BF16)
        for i in range(N // BQ):
            qi = q[i * BQ:(i + 1) * BQ, :]
            s = lax.dot_general(qi, k, _CONTRACT_LAST,
                                preferred_element_type=_F32) * SCALE
            m = jnp.max(s, axis=-1, keepdims=True)
            e = jnp.exp(s - m)
            pr = (e / jnp.sum(e, axis=-1, keepdims=True)).astype(_BF16)
            o_ref[i * BQ:(i + 1) * BQ, p * HD:(p + 1) * HD] = jnp.dot(
                pr, flat, preferred_element_type=_F32)


def kernel(x, proj_w, proj_b, wq, wk):
    x2d = x.reshape(N, D).astype(_BF16)
    b3d = proj_b.reshape(GH, 1, PAIR * HD)
    out = pl.pallas_call(
        _mha_kernel,
        grid=(GH,),
        in_specs=[
            pl.BlockSpec((N, D), lambda g: (0, 0)),
            pl.BlockSpec((PAIR * HD, D), lambda g: (g, 0)),
            pl.BlockSpec((1, 1, PAIR * HD), lambda g: (g, 0, 0)),
            pl.BlockSpec((PAIR, HD, HD), lambda g: (g, 0, 0)),
            pl.BlockSpec((PAIR, HD, HD), lambda g: (g, 0, 0)),
        ],
        out_specs=pl.BlockSpec((N, PAIR * HD), lambda g: (0, g)),
        out_shape=jax.ShapeDtypeStruct((N, D), jnp.float32),
        compiler_params=pltpu.CompilerParams(
            dimension_semantics=("parallel",),
            vmem_limit_bytes=56 * 1024 * 1024,
        ),
    )(x2d, proj_w.astype(_BF16), b3d, wq.astype(_BF16), wk.astype(_BF16))
    return out.reshape(B, S, D)


# speedup vs baseline: 1.6585x; 1.6585x over previous
import jax
import jax.numpy as jnp
from jax import lax
from jax.experimental import pallas as pl
from jax.experimental.pallas import tpu as pltpu

D = 1024
H = 16
HD = D // H
B, S = 2, 1024
N = B * S
PAIR = 2
GH = H // PAIR
BQ = 256
QSCALE = 1.4426950408889634 / 32.0

_CONTRACT_LAST = (((1,), (1,)), ((), ()))
_F32 = jnp.float32


def _mha_kernel(x_ref, w_ref, b_ref, wq_ref, wk_ref, o_ref):
    x = x_ref[...]
    w = w_ref[...]
    flat2 = lax.dot_general(x, w, _CONTRACT_LAST,
                            preferred_element_type=_F32) + b_ref[0]
    ones = jnp.ones((N, HD), dtype=_F32)
    for p in range(PAIR):
        flat = flat2[:, p * HD:(p + 1) * HD]
        vext = jnp.concatenate([flat, ones], axis=1)
        q = jnp.dot(flat, wq_ref[p], preferred_element_type=_F32) * QSCALE
        k = jnp.dot(flat, wk_ref[p], preferred_element_type=_F32)
        for i in range(N // BQ):
            qi = q[i * BQ:(i + 1) * BQ, :]
            s = lax.dot_general(qi, k, _CONTRACT_LAST,
                                preferred_element_type=_F32)
            m = jnp.max(s, axis=-1, keepdims=True)
            e = jnp.exp2(s - m)
            oe = jnp.dot(e, vext, preferred_element_type=_F32)
            inv = 1.0 / oe[:, HD:HD + 1]
            o_ref[i * BQ:(i + 1) * BQ, p * HD:(p + 1) * HD] = (
                oe[:, :HD] * inv)


def kernel(x, proj_w, proj_b, wq, wk):
    x2d = x.reshape(N, D)
    b3d = proj_b.reshape(GH, 1, PAIR * HD)
    out = pl.pallas_call(
        _mha_kernel,
        grid=(GH,),
        in_specs=[
            pl.BlockSpec((N, D), lambda g: (0, 0)),
            pl.BlockSpec((PAIR * HD, D), lambda g: (g, 0)),
            pl.BlockSpec((1, 1, PAIR * HD), lambda g: (g, 0, 0)),
            pl.BlockSpec((PAIR, HD, HD), lambda g: (g, 0, 0)),
            pl.BlockSpec((PAIR, HD, HD), lambda g: (g, 0, 0)),
        ],
        out_specs=pl.BlockSpec((N, PAIR * HD), lambda g: (0, g)),
        out_shape=jax.ShapeDtypeStruct((N, D), jnp.float32),
        compiler_params=pltpu.CompilerParams(
            dimension_semantics=("parallel",),
            vmem_limit_bytes=56 * 1024 * 1024,
        ),
    )(x2d, proj_w, b3d, wq, wk)
    return out.reshape(B, S, D)
